# baseline (speedup 1.0000x reference)
import jax
import jax.numpy as jnp
from jax import lax
from jax.experimental import pallas as pl
from jax.experimental.pallas import tpu as pltpu
from jax.experimental.pallas import tpu_sc as plsc

B, ROW, COL = 1024, 200, 128
NC, NS, L = 2, 16, 16
NW = NC * NS
IMGS_PER_W = B // NW
KCOL = COL // L


def _body(tensor_hbm, rp_hbm, cp_hbm, out_hbm,
          in_v0, in_v1, in_v2, out_v0, out_v1, rp_v, cp_v,
          sin0, sin1, sin2, sout0, sout1):
    wid = lax.axis_index("s") * NC + lax.axis_index("c")
    base_img = wid * IMGS_PER_W

    pltpu.sync_copy(rp_hbm, rp_v)
    pltpu.sync_copy(cp_hbm, cp_v)

    in_bufs, sins = (in_v0, in_v1, in_v2), (sin0, sin1, sin2)
    out_bufs, souts = (out_v0, out_v1), (sout0, sout1)

    cps = [cp_v[pl.ds(k * L, L)] for k in range(KCOL)]

    pltpu.async_copy(tensor_hbm.at[base_img], in_v0, sin0)
    pltpu.async_copy(tensor_hbm.at[base_img + 1], in_v1, sin1)

    for t in range(IMGS_PER_W):
        in_b, s_in = in_bufs[t % 3], sins[t % 3]
        out_b, s_out = out_bufs[t % 2], souts[t % 2]

        if t + 2 < IMGS_PER_W:
            pltpu.async_copy(tensor_hbm.at[base_img + t + 2],
                             in_bufs[(t + 2) % 3], sins[(t + 2) % 3])

        pltpu.make_async_copy(tensor_hbm.at[base_img + t], in_b, s_in).wait()

        if t >= 2:
            pltpu.make_async_copy(out_b, out_hbm.at[base_img + t - 2],
                                  s_out).wait()

        @plsc.parallel_loop(0, ROW, 1, unroll=4)
        def _(i):
            row_splat = plsc.load_gather(
                rp_v, [jnp.full((L,), i, jnp.int32)])
            for k in range(KCOL):
                x = plsc.load_gather(in_b, [row_splat, cps[k]])
                out_b[i, pl.ds(k * L, L)] = x

        pltpu.async_copy(out_b, out_hbm.at[base_img + t], s_out)

    pltpu.make_async_copy(out_v0, out_hbm.at[base_img + IMGS_PER_W - 2],
                          sout0).wait()
    pltpu.make_async_copy(out_v1, out_hbm.at[base_img + IMGS_PER_W - 1],
                          sout1).wait()


@jax.jit
def _permute(tensor, rowperm, colperm):
    kfn = pl.kernel(
        _body,
        out_type=jax.ShapeDtypeStruct((B, ROW, COL), jnp.float32),
        mesh=plsc.VectorSubcoreMesh(core_axis_name="c", subcore_axis_name="s"),
        compiler_params=pltpu.CompilerParams(needs_layout_passes=False),
        scratch_types=[
            pltpu.VMEM((ROW, COL), jnp.float32),
            pltpu.VMEM((ROW, COL), jnp.float32),
            pltpu.VMEM((ROW, COL), jnp.float32),
            pltpu.VMEM((ROW, COL), jnp.float32),
            pltpu.VMEM((ROW, COL), jnp.float32),
            pltpu.VMEM((ROW,), jnp.int32),
            pltpu.VMEM((COL,), jnp.int32),
            pltpu.SemaphoreType.DMA,
            pltpu.SemaphoreType.DMA,
            pltpu.SemaphoreType.DMA,
            pltpu.SemaphoreType.DMA,
            pltpu.SemaphoreType.DMA,
        ],
    )
    return kfn(tensor, rowperm, colperm)


def kernel(tensor, rowperm, colperm):
    return _permute(tensor, rowperm.astype(jnp.int32),
                    colperm.astype(jnp.int32))


# speedup vs baseline: 1.1105x; 1.1105x over previous
import jax
import jax.numpy as jnp
from jax import lax
from jax.experimental import pallas as pl
from jax.experimental.pallas import tpu as pltpu
from jax.experimental.pallas import tpu_sc as plsc

B, ROW, COL = 1024, 200, 128
NC, NS, L = 2, 16, 16
NW = NC * NS
IMGS_PER_W = B // NW
KCOL = COL // L


def _body(tensor_hbm, rp_hbm, cp_hbm, out_hbm,
          in_v0, in_v1, out_v0, out_v1, rp_v, cp_v,
          sin0, sin1, sout0, sout1):
    wid = lax.axis_index("s") * NC + lax.axis_index("c")
    base_img = wid * IMGS_PER_W

    pltpu.sync_copy(rp_hbm, rp_v)
    pltpu.sync_copy(cp_hbm, cp_v)

    in_bufs, out_bufs = (in_v0, in_v1), (out_v0, out_v1)
    sins, souts = (sin0, sin1), (sout0, sout1)

    cps = [cp_v[pl.ds(k * L, L)] for k in range(KCOL)]

    pltpu.async_copy(tensor_hbm.at[base_img], in_v0, sin0)

    def per_pair(p, _):
        for bslot in range(2):
            t = p * 2 + bslot
            in_b, out_b = in_bufs[bslot], out_bufs[bslot]
            s_in, s_out = sins[bslot], souts[bslot]

            @pl.when(t + 1 < IMGS_PER_W)
            def _():
                pltpu.async_copy(tensor_hbm.at[base_img + t + 1],
                                 in_bufs[1 - bslot], sins[1 - bslot])

            pltpu.make_async_copy(tensor_hbm.at[base_img + t], in_b,
                                  s_in).wait()

            @pl.when(t >= 2)
            def _():
                pltpu.make_async_copy(out_b, out_hbm.at[base_img + t - 2],
                                      s_out).wait()

            @plsc.parallel_loop(0, ROW, 1, unroll=4)
            def _(i):
                row_splat = plsc.load_gather(
                    rp_v, [jnp.full((L,), i, jnp.int32)])
                for k in range(KCOL):
                    x = plsc.load_gather(in_b, [row_splat, cps[k]])
                    out_b[i, pl.ds(k * L, L)] = x

            pltpu.async_copy(out_b, out_hbm.at[base_img + t], s_out)
        return 0

    lax.fori_loop(0, IMGS_PER_W // 2, per_pair, 0)

    pltpu.make_async_copy(out_v0, out_hbm.at[base_img + IMGS_PER_W - 2],
                          sout0).wait()
    pltpu.make_async_copy(out_v1, out_hbm.at[base_img + IMGS_PER_W - 1],
                          sout1).wait()


@jax.jit
def _permute(tensor, rowperm, colperm):
    kfn = pl.kernel(
        _body,
        out_type=jax.ShapeDtypeStruct((B, ROW, COL), jnp.float32),
        mesh=plsc.VectorSubcoreMesh(core_axis_name="c", subcore_axis_name="s"),
        compiler_params=pltpu.CompilerParams(needs_layout_passes=False),
        scratch_types=[
            pltpu.VMEM((ROW, COL), jnp.float32),
            pltpu.VMEM((ROW, COL), jnp.float32),
            pltpu.VMEM((ROW, COL), jnp.float32),
            pltpu.VMEM((ROW, COL), jnp.float32),
            pltpu.VMEM((ROW,), jnp.int32),
            pltpu.VMEM((COL,), jnp.int32),
            pltpu.SemaphoreType.DMA,
            pltpu.SemaphoreType.DMA,
            pltpu.SemaphoreType.DMA,
            pltpu.SemaphoreType.DMA,
        ],
    )
    return kfn(tensor, rowperm, colperm)


def kernel(tensor, rowperm, colperm):
    return _permute(tensor, rowperm.astype(jnp.int32),
                    colperm.astype(jnp.int32))
